# baseline (speedup 1.0000x reference)
import jax
import jax.numpy as jnp
from jax.experimental import pallas as pl

N = 10000
S = 32
D = 128
DOUT = 128
B = 80


def _dot(x, w):
    return jax.lax.dot_general(x, w, (((1,), (0,)), ((), ())),
                               preferred_element_type=jnp.float32)


def _fused_body(src_ref, sneg_ref, dst_ref, dneg_ref, w_ref,
                osrc_ref, odst_ref):
    w = w_ref[...]
    sagg = jnp.mean(jnp.reshape(sneg_ref[...], (B, S, D)), axis=1)
    dagg = jnp.mean(jnp.reshape(dneg_ref[...], (B, S, D)), axis=1)
    xs = jnp.concatenate([src_ref[...], sagg], axis=1)
    xd = jnp.concatenate([dst_ref[...], dagg], axis=1)
    osrc_ref[...] = jnp.maximum(_dot(xs, w), 0.0)
    odst_ref[...] = jnp.maximum(_dot(xd, w), 0.0)


@jax.jit
def kernel(src, src_neg, dst, dst_neg, w):
    grid = (N // B,)
    row_spec = pl.BlockSpec((B, D), lambda i: (i, 0))
    neg_spec = pl.BlockSpec((B * S, D), lambda i: (i, 0))
    w_spec = pl.BlockSpec((2 * D, DOUT), lambda i: (0, 0))
    out_spec = pl.BlockSpec((B, DOUT), lambda i: (i, 0))
    return pl.pallas_call(
        _fused_body,
        grid=grid,
        in_specs=[row_spec, neg_spec, row_spec, neg_spec, w_spec],
        out_specs=[out_spec, out_spec],
        out_shape=(jax.ShapeDtypeStruct((N, DOUT), jnp.float32),
                   jax.ShapeDtypeStruct((N, DOUT), jnp.float32)),
    )(src, src_neg, dst, dst_neg, w)


# speedup vs baseline: 1.3915x; 1.3915x over previous
import jax
import jax.numpy as jnp
from jax.experimental import pallas as pl

N = 10000
S = 32
D = 128
DOUT = 128
B = 200


def _dot(x, w):
    return jax.lax.dot_general(x, w, (((1,), (0,)), ((), ())),
                               preferred_element_type=jnp.float32)


def _fused_body(src_ref, sneg_ref, dst_ref, dneg_ref, w_ref,
                osrc_ref, odst_ref):
    w = w_ref[...]
    sagg = jnp.mean(jnp.reshape(sneg_ref[...], (B, S, D)), axis=1)
    dagg = jnp.mean(jnp.reshape(dneg_ref[...], (B, S, D)), axis=1)
    xs = jnp.concatenate([src_ref[...], sagg], axis=1)
    xd = jnp.concatenate([dst_ref[...], dagg], axis=1)
    osrc_ref[...] = jnp.maximum(_dot(xs, w), 0.0)
    odst_ref[...] = jnp.maximum(_dot(xd, w), 0.0)


@jax.jit
def kernel(src, src_neg, dst, dst_neg, w):
    grid = (N // B,)
    row_spec = pl.BlockSpec((B, D), lambda i: (i, 0))
    neg_spec = pl.BlockSpec((B * S, D), lambda i: (i, 0))
    w_spec = pl.BlockSpec((2 * D, DOUT), lambda i: (0, 0))
    out_spec = pl.BlockSpec((B, DOUT), lambda i: (i, 0))
    return pl.pallas_call(
        _fused_body,
        grid=grid,
        in_specs=[row_spec, neg_spec, row_spec, neg_spec, w_spec],
        out_specs=[out_spec, out_spec],
        out_shape=(jax.ShapeDtypeStruct((N, DOUT), jnp.float32),
                   jax.ShapeDtypeStruct((N, DOUT), jnp.float32)),
    )(src, src_neg, dst, dst_neg, w)
